# baseline (speedup 1.0000x reference)
import functools

import jax
import jax.numpy as jnp
from jax import lax
from jax.experimental import pallas as pl
from jax.experimental.pallas import tpu as pltpu
from jax.experimental.pallas import tpu_sc as plsc

VOCAB = 100000
EMBED_DIM = 64
HIDDEN = 128
L = 200

VB = 4000
NB = VOCAB // VB


_G = 2
_GL = L // _G


@functools.cache
def _sc_bag_sum_kernel():
    mesh = plsc.VectorSubcoreMesh(core_axis_name="c", subcore_axis_name="s")
    return pl.kernel(
        _sc_bag_sum,
        out_type=jax.ShapeDtypeStruct((1, EMBED_DIM), jnp.float32),
        mesh=mesh,
        scratch_types=[
            pltpu.VMEM((_G, _GL), jnp.int32),
            pltpu.VMEM((_G, _GL, EMBED_DIM), jnp.float32),
            pltpu.VMEM((1, EMBED_DIM), jnp.float32),
            pltpu.SemaphoreType.DMA,
        ],
        compiler_params=pltpu.CompilerParams(use_tc_tiling_on_sc=False),
    )


def _sc_bag_sum(idx_hbm, emb_hbm, out_hbm, idx_v, rows_v, sum_v, sem):
    cid = lax.axis_index("c")
    sid = lax.axis_index("s")

    @pl.when(jnp.logical_and(cid == 0, sid == 0))
    def _():
        pltpu.sync_copy(idx_hbm, idx_v)
        cp0 = pltpu.make_async_copy(emb_hbm.at[idx_v.at[0]], rows_v.at[0], sem)
        cp1 = pltpu.make_async_copy(emb_hbm.at[idx_v.at[1]], rows_v.at[1], sem)
        cp0.start()
        cp1.start()
        cp0.wait()
        cp1.wait()

        nvec = EMBED_DIM // 16

        def body(i, accs):
            g = i // _GL
            r = i % _GL
            return tuple(
                accs[c] + rows_v[g, r, pl.ds(c * 16, 16)] for c in range(nvec)
            )

        accs = tuple(jnp.zeros((16,), jnp.float32) for _ in range(nvec))
        accs = lax.fori_loop(0, L, body, accs)
        for c in range(nvec):
            sum_v[0, pl.ds(c * 16, 16)] = accs[c]
        pltpu.sync_copy(sum_v, out_hbm)


def _tc_body(sum_ref, w1_ref, b1_ref, w2_ref, b2_ref, out_ref, h_ref):
    j = pl.program_id(0)

    @pl.when(j == 0)
    def _():
        pre = lax.dot_general(
            sum_ref[...], w1_ref[...],
            (((1,), (1,)), ((), ())),
            preferred_element_type=jnp.float32,
        ) + b1_ref[...]
        h_ref[...] = jnp.maximum(pre, 0.0)

    logits = lax.dot_general(
        h_ref[...], w2_ref[...],
        (((1,), (1,)), ((), ())),
        preferred_element_type=jnp.float32,
    ) + b2_ref[0]
    out_ref[pl.ds(j, 1), :] = logits

    @pl.when(j == NB - 1)
    def _():
        x = out_ref[...]
        m = jnp.max(x)
        lse = m + jnp.log(jnp.sum(jnp.exp(x - m)))
        out_ref[...] = x - lse


_tc_call = pl.pallas_call(
    _tc_body,
    grid=(NB,),
    in_specs=[
        pl.BlockSpec((1, EMBED_DIM), lambda j: (0, 0)),
        pl.BlockSpec((HIDDEN, EMBED_DIM), lambda j: (0, 0)),
        pl.BlockSpec((1, HIDDEN), lambda j: (0, 0)),
        pl.BlockSpec((VB, HIDDEN), lambda j: (j, 0)),
        pl.BlockSpec((1, 1, VB), lambda j: (j, 0, 0)),
    ],
    out_specs=pl.BlockSpec((NB, VB), lambda j: (0, 0)),
    out_shape=jax.ShapeDtypeStruct((NB, VB), jnp.float32),
    scratch_shapes=[pltpu.VMEM((1, HIDDEN), jnp.float32)],
    compiler_params=pltpu.CompilerParams(
        dimension_semantics=("arbitrary",),
    ),
)


def kernel(inputs, emb, W1, b1, W2, b2):
    idx = inputs.astype(jnp.int32).reshape(_G, _GL)
    bag = _sc_bag_sum_kernel()(idx, emb)
    out = _tc_call(bag, W1, b1.reshape(1, HIDDEN), W2, b2.reshape(NB, 1, VB))
    return out.reshape(1, VOCAB)


# speedup vs baseline: 1.3154x; 1.3154x over previous
import functools

import jax
import jax.numpy as jnp
from jax import lax
from jax.experimental import pallas as pl
from jax.experimental.pallas import tpu as pltpu
from jax.experimental.pallas import tpu_sc as plsc

VOCAB = 100000
EMBED_DIM = 64
HIDDEN = 128
L = 200

VB = 4000
NB = VOCAB // VB


_LP = 208


@functools.cache
def _sc_bag_sum_kernel():
    mesh = plsc.VectorSubcoreMesh(core_axis_name="c", subcore_axis_name="s")
    return pl.kernel(
        _sc_bag_sum,
        out_type=jax.ShapeDtypeStruct((1, EMBED_DIM), jnp.float32),
        mesh=mesh,
        scratch_types=[
            pltpu.VMEM((_LP,), jnp.int32),
            pltpu.VMEM((_LP, EMBED_DIM), jnp.float32),
            pltpu.VMEM((1, EMBED_DIM), jnp.float32),
            pltpu.SemaphoreType.DMA,
        ],
    )


def _sc_bag_sum(idx_hbm, emb_hbm, out_hbm, idx_v, rows_v, sum_v, sem):
    cid = lax.axis_index("c")
    sid = lax.axis_index("s")

    @pl.when(jnp.logical_and(cid == 0, sid == 0))
    def _():
        pltpu.sync_copy(idx_hbm, idx_v.at[pl.ds(0, L)])
        tail = idx_v[pl.ds(_LP - 16, 16)]
        lane = lax.iota(jnp.int32, 16)
        idx_v[pl.ds(_LP - 16, 16)] = jnp.where(lane < 16 - (_LP - L), tail, 0)

        def fire(g, _):
            vrow = idx_v[pl.ds(g * 16, 16)]
            for k in range(16):
                pltpu.make_async_copy(
                    emb_hbm.at[pl.ds(vrow[k], 1), :],
                    rows_v.at[pl.ds(g * 16 + k, 1), :],
                    sem,
                ).start()
            return 0

        lax.fori_loop(0, _LP // 16, fire, 0)

        def drain(i, _):
            pltpu.make_async_copy(
                emb_hbm.at[pl.ds(0, 1), :], rows_v.at[pl.ds(0, 1), :], sem
            ).wait()
            return 0

        lax.fori_loop(0, _LP, drain, 0)

        nvec = EMBED_DIM // 16

        def body(i, accs):
            return tuple(
                accs[c] + rows_v[i, pl.ds(c * 16, 16)] for c in range(nvec)
            )

        accs = tuple(jnp.zeros((16,), jnp.float32) for _ in range(nvec))
        accs = lax.fori_loop(0, L, body, accs)
        for c in range(nvec):
            sum_v[0, pl.ds(c * 16, 16)] = accs[c]
        pltpu.sync_copy(sum_v, out_hbm)


def _tc_body(sum_ref, w1_ref, b1_ref, w2_ref, b2_ref, out_ref, h_ref):
    j = pl.program_id(0)

    @pl.when(j == 0)
    def _():
        pre = lax.dot_general(
            sum_ref[...], w1_ref[...],
            (((1,), (1,)), ((), ())),
            preferred_element_type=jnp.float32,
        ) + b1_ref[...]
        h_ref[...] = jnp.maximum(pre, 0.0)

    logits = lax.dot_general(
        h_ref[...], w2_ref[...],
        (((1,), (1,)), ((), ())),
        preferred_element_type=jnp.float32,
    ) + b2_ref[0]
    out_ref[pl.ds(j, 1), :] = logits

    @pl.when(j == NB - 1)
    def _():
        x = out_ref[...]
        m = jnp.max(x)
        lse = m + jnp.log(jnp.sum(jnp.exp(x - m)))
        out_ref[...] = x - lse


_tc_call = pl.pallas_call(
    _tc_body,
    grid=(NB,),
    in_specs=[
        pl.BlockSpec((1, EMBED_DIM), lambda j: (0, 0)),
        pl.BlockSpec((HIDDEN, EMBED_DIM), lambda j: (0, 0)),
        pl.BlockSpec((1, HIDDEN), lambda j: (0, 0)),
        pl.BlockSpec((VB, HIDDEN), lambda j: (j, 0)),
        pl.BlockSpec((1, 1, VB), lambda j: (j, 0, 0)),
    ],
    out_specs=pl.BlockSpec((NB, VB), lambda j: (0, 0)),
    out_shape=jax.ShapeDtypeStruct((NB, VB), jnp.float32),
    scratch_shapes=[pltpu.VMEM((1, HIDDEN), jnp.float32)],
    compiler_params=pltpu.CompilerParams(
        dimension_semantics=("arbitrary",),
    ),
)


def kernel(inputs, emb, W1, b1, W2, b2):
    idx = inputs.astype(jnp.int32)
    bag = _sc_bag_sum_kernel()(idx, emb)
    out = _tc_call(bag, W1, b1.reshape(1, HIDDEN), W2, b2.reshape(NB, 1, VB))
    return out.reshape(1, VOCAB)
